# baseline (speedup 1.0000x reference)
import functools

import jax
import jax.numpy as jnp
from jax.experimental import pallas as pl
from jax.experimental.pallas import tpu as pltpu

B, S, D, H = 1, 4096, 768, 12
DH = D // H
W = 256
NC = S // W
NEG = -1e9


def _fused_kernel(xq_ref, xn_ref, wq_ref, bq_ref, wkv_ref, bkv_ref,
                  mwin_ref, qmask_ref, o_ref, k_sc, v_sc):
    c = pl.program_id(0)

    def project_kv(x, row_start):
        kv = jax.lax.dot_general(
            x, wkv_ref[...], (((1,), (0,)), ((), ())),
            preferred_element_type=jnp.float32) + bkv_ref[...]
        k_sc[pl.ds(row_start, W), :] = kv[:, :D]
        v_sc[pl.ds(row_start, W), :] = kv[:, D:]

    @pl.when(c == 0)
    def _warmup():
        project_kv(xq_ref[...], 0)
        k_sc[pl.ds(2 * W, W), :] = jnp.zeros((W, D), jnp.float32)
        v_sc[pl.ds(2 * W, W), :] = jnp.zeros((W, D), jnp.float32)

    @pl.when(c < NC - 1)
    def _next_kv():
        project_kv(xn_ref[...], (c + 1) * W)

    q = jax.lax.dot_general(
        xq_ref[...], wq_ref[...], (((1,), (0,)), ((), ())),
        preferred_element_type=jnp.float32) + bq_ref[...]

    start = pl.multiple_of(jnp.clip(c - 1, 0, NC - 3) * W, W)
    row = jax.lax.broadcasted_iota(jnp.int32, (W, 3 * W), 0)
    col = jax.lax.broadcasted_iota(jnp.int32, (W, 3 * W), 1)
    rel = (start + col) - (c * W + row)
    valid = (rel >= -W) & (rel <= W)
    addmask = jnp.where(valid, mwin_ref[0, 0][None, :], NEG)
    notmasked = (1.0 - qmask_ref[0])[:, None]

    outs = []
    for h in range(H):
        sl = slice(h * DH, (h + 1) * DH)
        qh = q[:, sl]
        kwin = k_sc[pl.ds(start, 3 * W), sl]
        vwin = v_sc[pl.ds(start, 3 * W), sl]
        scores = jax.lax.dot_general(
            qh, kwin, (((1,), (1,)), ((), ())),
            preferred_element_type=jnp.float32)
        e = jnp.exp(scores + addmask)
        denom = jnp.sum(e, axis=-1, keepdims=True)
        oh = jax.lax.dot_general(
            e, vwin, (((1,), (0,)), ((), ())),
            preferred_element_type=jnp.float32) / denom
        outs.append(oh)
    o_ref[...] = jnp.concatenate(outs, axis=1) * notmasked


@functools.partial(jax.jit, static_argnames=())
def kernel(hidden_states, attention_mask, is_index_masked, is_index_global_attn,
           is_global_attn, Wq, bq, Wk, bk, Wv, bv):
    x = hidden_states.reshape(S, D)
    sc = jnp.sqrt(jnp.float32(DH))
    wq = Wq / sc
    bqv = (bq / sc).reshape(1, D)
    wkv = jnp.concatenate([Wk, Wv], axis=1)
    bkv = jnp.concatenate([bk, bv]).reshape(1, 2 * D)

    starts = jnp.clip((jnp.arange(NC) - 1) * W, 0, S - 3 * W)
    gidx = jnp.arange(3 * W)[None, :] + starts[:, None]
    mwin = attention_mask.reshape(S)[gidx].reshape(NC, 1, 3 * W)
    qmask = is_index_masked.astype(jnp.float32).reshape(1, S)

    out = pl.pallas_call(
        _fused_kernel,
        grid=(NC,),
        in_specs=[
            pl.BlockSpec((W, D), lambda c: (c, 0)),
            pl.BlockSpec((W, D), lambda c: (jnp.minimum(c + 1, NC - 1), 0)),
            pl.BlockSpec((D, D), lambda c: (0, 0)),
            pl.BlockSpec((1, D), lambda c: (0, 0)),
            pl.BlockSpec((D, 2 * D), lambda c: (0, 0)),
            pl.BlockSpec((1, 2 * D), lambda c: (0, 0)),
            pl.BlockSpec((1, 1, 3 * W), lambda c: (c, 0, 0)),
            pl.BlockSpec((1, W), lambda c: (0, c)),
        ],
        out_specs=pl.BlockSpec((W, D), lambda c: (c, 0)),
        out_shape=jax.ShapeDtypeStruct((S, D), jnp.float32),
        scratch_shapes=[
            pltpu.VMEM((S, D), jnp.float32),
            pltpu.VMEM((S, D), jnp.float32),
        ],
    )(x, x, wq, bqv, wkv, bkv, mwin, qmask)

    return out.reshape(B, S, D)


# speedup vs baseline: 2.8718x; 2.8718x over previous
import functools

import jax
import jax.numpy as jnp
from jax.experimental import pallas as pl
from jax.experimental.pallas import tpu as pltpu

B, S, D, H = 1, 4096, 768, 12
DH = D // H
W = 256
NC = S // W
NEG = -1e9
QSCALE = 0.125


def _qkv_proj_kernel(x_ref, wq_ref, wk_ref, wv_ref, b_ref, o_ref):
    x = x_ref[...]
    for i, w_ref in enumerate((wq_ref, wk_ref, wv_ref)):
        o_ref[:, i * D:(i + 1) * D] = jax.lax.dot_general(
            x, w_ref[...], (((1,), (0,)), ((), ())),
            preferred_element_type=jnp.float32) + b_ref[:, i * D:(i + 1) * D]


def _attn_kernel(q_ref, k0_ref, k1_ref, k2_ref, v0_ref, v1_ref, v2_ref,
                 m0_ref, m1_ref, m2_ref, qmask_ref, o_ref):
    c = pl.program_id(0)
    row = jax.lax.broadcasted_iota(jnp.int32, (W, W), 0)
    col = jax.lax.broadcasted_iota(jnp.int32, (W, W), 1)
    valid = jnp.concatenate(
        [((lambda rel: (rel >= -W) & (rel <= W))((d - 1) * W + col - row)
          & (0 <= c + d - 1) & (c + d - 1 < NC))
         for d in range(3)], axis=1)
    mvec = jnp.concatenate([m0_ref[0], m1_ref[0], m2_ref[0]])
    addmask = jnp.where(valid, mvec[None, :], NEG)
    notmasked = (1.0 - qmask_ref[0])[:, None]
    outs = []
    for h in range(H):
        sl = slice(h * DH, (h + 1) * DH)
        qh = q_ref[:, sl]
        kcat = jnp.concatenate([k0_ref[:, sl], k1_ref[:, sl], k2_ref[:, sl]],
                               axis=0)
        vcat = jnp.concatenate([v0_ref[:, sl], v1_ref[:, sl], v2_ref[:, sl]],
                               axis=0)
        scores = jax.lax.dot_general(
            qh, kcat, (((1,), (1,)), ((), ())),
            preferred_element_type=jnp.float32)
        e = jnp.exp(scores * QSCALE + addmask)
        denom = jnp.sum(e, axis=-1, keepdims=True)
        oh = jax.lax.dot_general(
            e, vcat, (((1,), (0,)), ((), ())),
            preferred_element_type=jnp.float32) / denom
        outs.append(oh)
    o_ref[...] = jnp.concatenate(outs, axis=1) * notmasked


@functools.partial(jax.jit, static_argnames=())
def kernel(hidden_states, attention_mask, is_index_masked, is_index_global_attn,
           is_global_attn, Wq, bq, Wk, bk, Wv, bv):
    x = hidden_states.reshape(S, D)
    b = jnp.concatenate([bq, bk, bv]).reshape(1, 3 * D)

    rows = 512
    qkv = pl.pallas_call(
        _qkv_proj_kernel,
        grid=(S // rows,),
        in_specs=[
            pl.BlockSpec((rows, D), lambda i: (i, 0)),
            pl.BlockSpec((D, D), lambda i: (0, 0)),
            pl.BlockSpec((D, D), lambda i: (0, 0)),
            pl.BlockSpec((D, D), lambda i: (0, 0)),
            pl.BlockSpec((1, 3 * D), lambda i: (0, 0)),
        ],
        out_specs=pl.BlockSpec((rows, 3 * D), lambda i: (i, 0)),
        out_shape=jax.ShapeDtypeStruct((S, 3 * D), jnp.float32),
        compiler_params=pltpu.CompilerParams(
            dimension_semantics=("parallel",)),
    )(x, Wq, Wk, Wv, b)

    amask = attention_mask.reshape(1, S)
    qmask = is_index_masked.astype(jnp.float32).reshape(1, S)

    def prev_c(c):
        return jnp.maximum(c - 1, 0)

    def next_c(c):
        return jnp.minimum(c + 1, NC - 1)

    out = pl.pallas_call(
        _attn_kernel,
        grid=(NC,),
        in_specs=[
            pl.BlockSpec((W, D), lambda c: (c, 0)),
            pl.BlockSpec((W, D), lambda c: (prev_c(c), 1)),
            pl.BlockSpec((W, D), lambda c: (c, 1)),
            pl.BlockSpec((W, D), lambda c: (next_c(c), 1)),
            pl.BlockSpec((W, D), lambda c: (prev_c(c), 2)),
            pl.BlockSpec((W, D), lambda c: (c, 2)),
            pl.BlockSpec((W, D), lambda c: (next_c(c), 2)),
            pl.BlockSpec((1, W), lambda c: (0, prev_c(c))),
            pl.BlockSpec((1, W), lambda c: (0, c)),
            pl.BlockSpec((1, W), lambda c: (0, next_c(c))),
            pl.BlockSpec((1, W), lambda c: (0, c)),
        ],
        out_specs=pl.BlockSpec((W, D), lambda c: (c, 0)),
        out_shape=jax.ShapeDtypeStruct((S, D), jnp.float32),
        compiler_params=pltpu.CompilerParams(
            dimension_semantics=("parallel",)),
    )(qkv, qkv, qkv, qkv, qkv, qkv, qkv, amask, amask, amask, qmask)

    return out.reshape(B, S, D)
